# baseline (speedup 1.0000x reference)
import jax
import jax.numpy as jnp
from jax.experimental import pallas as pl
from jax.experimental.pallas import tpu as pltpu

_MASK_RATIO = 0.75


def _fused_kernel(noise_ref, x_ref, o_ref, mt_ref, *, k):
    @pl.when(pl.program_id(0) == 0)
    def _compute_mask():
        bits = jax.lax.bitcast_convert_type(noise_ref[...], jnp.int32).T
        l, d = bits.shape

        def body(_, carry):
            lo, hi = carry
            mid = lo + (hi - lo) // 2
            cnt = jnp.sum((bits <= mid).astype(jnp.int32), axis=0,
                          keepdims=True)
            ge = cnt >= k
            return jnp.where(ge, lo, mid + 1), jnp.where(ge, mid, hi)

        lo = jnp.zeros((1, d), jnp.int32)
        hi = jnp.full((1, d), 1 << 30, jnp.int32)
        lo, hi = jax.lax.fori_loop(0, 30, body, (lo, hi))
        thresh = lo

        lt = bits < thresh
        eq = bits == thresh
        cnt_lt = jnp.sum(lt.astype(jnp.int32), axis=0, keepdims=True)
        cnt_eq = jnp.sum(eq.astype(jnp.int32), axis=0, keepdims=True)
        ties_to_keep = (k - cnt_lt).astype(jnp.float32)

        mt_ref[...] = (lt | eq).astype(jnp.float32)

        @pl.when(jnp.any(cnt_eq + cnt_lt > k))
        def _stable_tie_break():
            row = jax.lax.broadcasted_iota(jnp.int32, (l, l), 0)
            col = jax.lax.broadcasted_iota(jnp.int32, (l, l), 1)
            stri = (col < row).astype(jnp.float32)
            prefix_eq = jax.lax.dot(stri, eq.astype(jnp.float32),
                                    preferred_element_type=jnp.float32)
            keep = lt | (eq & (prefix_eq < ties_to_keep))
            mt_ref[...] = keep.astype(jnp.float32)

    xb = x_ref[...]
    r, d = xb.shape
    l = mt_ref.shape[0]
    xb3 = xb.reshape(r // l, l, d)
    o_ref[...] = (xb3 * mt_ref[...][None]).reshape(r, d)


def kernel(x, noise):
    d, c, h, w = x.shape
    l = h * w
    k = int(l * (1 - _MASK_RATIO))
    x2 = jnp.transpose(x, (1, 2, 3, 0)).reshape(c * l, d)

    blk = 24 * l
    out2 = pl.pallas_call(
        lambda nr, xr, orf, mt: _fused_kernel(nr, xr, orf, mt, k=k),
        grid=(c * l // blk,),
        in_specs=[
            pl.BlockSpec((d, l), lambda i: (0, 0)),
            pl.BlockSpec((blk, d), lambda i: (i, 0)),
        ],
        out_specs=pl.BlockSpec((blk, d), lambda i: (i, 0)),
        out_shape=jax.ShapeDtypeStruct((c * l, d), x.dtype),
        scratch_shapes=[pltpu.VMEM((l, d), jnp.float32)],
    )(noise, x2)

    return out2.reshape(c, h, w, d).transpose(3, 0, 1, 2)
